# baseline (speedup 1.0000x reference)
import functools

import jax
import jax.numpy as jnp
from jax import lax
from jax.experimental import pallas as pl
from jax.experimental.pallas import tpu as pltpu
from jax.experimental.pallas import tpu_sc as plsc

_NUM_CORES = 2
_NUM_SUBCORES = 16
_NUM_WORKERS = _NUM_CORES * _NUM_SUBCORES

_HIDDEN = 64
_TOKENS_PER_BIN = 20
_GROUP_BINS = 32
_GROUP_ROWS = _GROUP_BINS * _TOKENS_PER_BIN
_IDX_MINOR = 128
_IDX_CHUNKS = _GROUP_ROWS // _IDX_MINOR


def _make_kernel(num_bins):
  assert num_bins % (_NUM_WORKERS * _GROUP_BINS) == 0
  bins_per_worker = num_bins // _NUM_WORKERS
  num_groups = bins_per_worker // _GROUP_BINS
  tok_rows_per_group = _GROUP_ROWS // _IDX_MINOR
  h_chunks = _HIDDEN // 16

  mesh = plsc.VectorSubcoreMesh(
      core_axis_name="c",
      subcore_axis_name="s",
      num_cores=_NUM_CORES,
      num_subcores=_NUM_SUBCORES,
  )

  @functools.partial(
      pl.kernel,
      mesh=mesh,
      compiler_params=pltpu.CompilerParams(
          needs_layout_passes=False, use_tc_tiling_on_sc=False),
      out_type=jax.ShapeDtypeStruct((num_bins * _HIDDEN, ), jnp.float32),
      scratch_types=[
          pltpu.VMEM((_GROUP_ROWS, ), jnp.int32),
          pltpu.VMEM((_GROUP_ROWS, _HIDDEN), jnp.float32),
          pltpu.VMEM((_GROUP_BINS * _HIDDEN, ), jnp.float32),
          pltpu.SemaphoreType.DMA,
      ],
  )
  def embed(tok_flat_hbm, table_hbm, out_hbm, tokf_v, rows_v, out_v, sem):
    wid = lax.axis_index("s") * _NUM_CORES + lax.axis_index("c")
    worker_bin0 = wid * bins_per_worker
    lane = lax.iota(jnp.int32, 16)

    def group_body(g, carry):
      bin0 = worker_bin0 + g * _GROUP_BINS
      pltpu.sync_copy(
          tok_flat_hbm.at[pl.ds(bin0 * _TOKENS_PER_BIN, _GROUP_ROWS)],
          tokf_v,
      )
      copies = [
          pltpu.async_copy(
              table_hbm.at[tokf_v.at[pl.ds(j * _IDX_MINOR, _IDX_MINOR)]],
              rows_v.at[pl.ds(j * _IDX_MINOR, _IDX_MINOR)],
              sem,
          )
          for j in range(_IDX_CHUNKS)
      ]
      for cp in copies:
        cp.wait()

      def bin_body(b, carry2):
        row0 = b * _TOKENS_PER_BIN
        acc = [rows_v[row0, pl.ds(h * 16, 16)] for h in range(h_chunks)]
        for l in range(1, _TOKENS_PER_BIN):
          r = row0 + l
          for h in range(h_chunks):
            acc[h] = acc[h] + rows_v[r, pl.ds(h * 16, 16)]
        for h in range(h_chunks):
          out_v[pl.ds(b * _HIDDEN + h * 16, 16)] = acc[h]
        return carry2

      lax.fori_loop(0, _GROUP_BINS, bin_body, 0)

      for half in range(_GROUP_BINS // 16):
        binv = lane + half * 16
        cnt = jnp.zeros((16,), jnp.int32)
        for l in range(_TOKENS_PER_BIN):
          tv = plsc.load_gather(tokf_v, [binv * _TOKENS_PER_BIN + l])
          cnt = cnt + jnp.where(tv != 0, jnp.int32(1), jnp.int32(0))
        inv = 1.0 / jnp.maximum(cnt, 1).astype(jnp.float32)

        def col_body(d, carry3, binv=binv, inv=inv):
          idx = binv * _HIDDEN + d
          col = plsc.load_gather(out_v, [idx])
          plsc.store_scatter(out_v, [idx], col * inv)
          return carry3

        lax.fori_loop(0, _HIDDEN, col_body, 0)

      pltpu.sync_copy(
          out_v, out_hbm.at[pl.ds(bin0 * _HIDDEN, _GROUP_BINS * _HIDDEN)])
      return carry

    lax.fori_loop(0, num_groups, group_body, 0)

  return embed


def kernel(tokens, table):
  assert tokens.ndim == 3 and table.ndim == 2
  batch, feats, tpb = tokens.shape
  assert tpb == _TOKENS_PER_BIN and table.shape[1] == _HIDDEN
  num_bins = batch * feats
  tok_flat = jnp.maximum(tokens.astype(jnp.int32), 0).reshape(-1)
  out = _make_kernel(num_bins)(tok_flat, table)
  return out.reshape(batch, feats, _HIDDEN)


# speedup vs baseline: 1.2370x; 1.2370x over previous
import functools

import jax
import jax.numpy as jnp
from jax import lax
from jax.experimental import pallas as pl
from jax.experimental.pallas import tpu as pltpu
from jax.experimental.pallas import tpu_sc as plsc

_NUM_CORES = 2
_NUM_SUBCORES = 16
_NUM_WORKERS = _NUM_CORES * _NUM_SUBCORES

_HIDDEN = 64
_TOKENS_PER_BIN = 20
_GROUP_BINS = 32
_GROUP_ROWS = _GROUP_BINS * _TOKENS_PER_BIN
_IDX_MINOR = 128
_IDX_CHUNKS = _GROUP_ROWS // _IDX_MINOR
_H_CHUNKS = _HIDDEN // 16


def _make_kernel(num_bins):
  assert num_bins % (_NUM_WORKERS * _GROUP_BINS) == 0
  bins_per_worker = num_bins // _NUM_WORKERS
  num_groups = bins_per_worker // _GROUP_BINS
  assert num_groups % 2 == 0 and num_groups >= 4

  mesh = plsc.VectorSubcoreMesh(
      core_axis_name="c",
      subcore_axis_name="s",
      num_cores=_NUM_CORES,
      num_subcores=_NUM_SUBCORES,
  )

  @functools.partial(
      pl.kernel,
      mesh=mesh,
      compiler_params=pltpu.CompilerParams(
          needs_layout_passes=False, use_tc_tiling_on_sc=False),
      out_type=jax.ShapeDtypeStruct((num_bins * _HIDDEN, ), jnp.float32),
      scratch_types=[
          pltpu.VMEM((_GROUP_ROWS, ), jnp.int32),
          pltpu.VMEM((_GROUP_ROWS, ), jnp.int32),
          pltpu.VMEM((_GROUP_ROWS, _HIDDEN), jnp.float32),
          pltpu.VMEM((_GROUP_ROWS, _HIDDEN), jnp.float32),
          pltpu.VMEM((_GROUP_BINS * _HIDDEN, ), jnp.float32),
          pltpu.SemaphoreType.DMA,
          pltpu.SemaphoreType.DMA,
          pltpu.SemaphoreType.DMA,
          pltpu.SemaphoreType.DMA,
      ],
  )
  def embed(tok_hbm, table_hbm, out_hbm, tok_v0, tok_v1, rows_v0, rows_v1,
            out_v, tok_sem0, tok_sem1, rows_sem0, rows_sem1):
    wid = lax.axis_index("s") * _NUM_CORES + lax.axis_index("c")
    worker_tok0 = wid * bins_per_worker * _TOKENS_PER_BIN
    worker_out0 = wid * bins_per_worker * _HIDDEN
    lane = lax.iota(jnp.int32, 16)
    tok_refs = (tok_v0, tok_v1)
    rows_refs = (rows_v0, rows_v1)
    tok_sems = (tok_sem0, tok_sem1)
    rows_sems = (rows_sem0, rows_sem1)

    def fire_tok(k, buf):
      src = tok_hbm.at[pl.ds(worker_tok0 + k * _GROUP_ROWS, _GROUP_ROWS)]
      pltpu.async_copy(src, tok_refs[buf], tok_sems[buf])

    def drain_tok(buf):
      pltpu.make_async_copy(
          tok_hbm.at[pl.ds(0, _GROUP_ROWS)], tok_refs[buf],
          tok_sems[buf]).wait()

    def fire_gathers(buf):
      for j in range(_IDX_CHUNKS):
        pltpu.async_copy(
            table_hbm.at[tok_refs[buf].at[pl.ds(j * _IDX_MINOR, _IDX_MINOR)]],
            rows_refs[buf].at[pl.ds(j * _IDX_MINOR, _IDX_MINOR)],
            rows_sems[buf],
        )

    def drain_rows(buf):
      pltpu.make_async_copy(
          table_hbm.at[pl.ds(0, _GROUP_ROWS)], rows_refs[buf],
          rows_sems[buf]).wait()

    def phase(g, cur, fire_next_tok, fire_next_gather):
      nxt = 1 - cur
      rows_cur = rows_refs[cur]
      drain_rows(cur)
      if fire_next_gather:
        drain_tok(nxt)
      invs = []
      for h2 in range(_GROUP_BINS // 16):
        binv = lane + h2 * 16
        cnt = jnp.zeros((16,), jnp.int32)
        for l in range(_TOKENS_PER_BIN):
          tv = plsc.load_gather(
              tok_refs[cur], [binv * _TOKENS_PER_BIN + l])
          cnt = cnt + jnp.where(tv != 0, jnp.int32(1), jnp.int32(0))
        invs.append(1.0 / jnp.maximum(cnt, 1).astype(jnp.float32))
      if fire_next_tok:
        fire_tok(g + 2, cur)
      if fire_next_gather:
        fire_gathers(nxt)

      def bin_body(b, carry):
        row0 = b * _TOKENS_PER_BIN
        acc = [rows_cur[row0, pl.ds(h * 16, 16)] for h in range(_H_CHUNKS)]
        for l in range(1, _TOKENS_PER_BIN):
          r = row0 + l
          for h in range(_H_CHUNKS):
            acc[h] = acc[h] + rows_cur[r, pl.ds(h * 16, 16)]
        for h in range(_H_CHUNKS):
          out_v[pl.ds(b * _HIDDEN + h * 16, 16)] = acc[h]
        return carry

      lax.fori_loop(0, _GROUP_BINS, bin_body, 0)

      for h2 in range(_GROUP_BINS // 16):
        base = (lane + h2 * 16) * _HIDDEN
        for d in range(_HIDDEN):
          idx = base + d
          col = plsc.load_gather(out_v, [idx])
          plsc.store_scatter(out_v, [idx], col * invs[h2])

      pltpu.sync_copy(
          out_v,
          out_hbm.at[pl.ds(worker_out0 + g * _GROUP_BINS * _HIDDEN,
                           _GROUP_BINS * _HIDDEN)],
      )

    fire_tok(0, 0)
    fire_tok(1, 1)
    drain_tok(0)
    fire_gathers(0)

    def pair_body(g2, carry):
      g = g2 * 2
      phase(g, 0, True, True)
      phase(g + 1, 1, True, True)
      return carry

    lax.fori_loop(0, num_groups // 2 - 1, pair_body, 0)
    phase(num_groups - 2, 0, False, True)
    phase(num_groups - 1, 1, False, False)

  return embed


def kernel(tokens, table):
  assert tokens.ndim == 3 and table.ndim == 2
  batch, feats, tpb = tokens.shape
  assert tpb == _TOKENS_PER_BIN and table.shape[1] == _HIDDEN
  num_bins = batch * feats
  tok_flat = tokens.astype(jnp.int32).reshape(-1)
  out = _make_kernel(num_bins)(tok_flat, table)
  return out.reshape(batch, feats, _HIDDEN)


# speedup vs baseline: 1.4944x; 1.2081x over previous
import functools

import jax
import jax.numpy as jnp
from jax import lax
from jax.experimental import pallas as pl
from jax.experimental.pallas import tpu as pltpu
from jax.experimental.pallas import tpu_sc as plsc

_NUM_CORES = 2
_NUM_SUBCORES = 16
_NUM_WORKERS = _NUM_CORES * _NUM_SUBCORES

_HIDDEN = 64
_TOKENS_PER_BIN = 20
_GROUP_BINS = 32
_GROUP_ROWS = _GROUP_BINS * _TOKENS_PER_BIN
_IDX_MINOR = 128
_IDX_CHUNKS = _GROUP_ROWS // _IDX_MINOR
_H_CHUNKS = _HIDDEN // 16

_GATHER_DNUMS = lax.GatherDimensionNumbers(
    offset_dims=(), collapsed_slice_dims=(0,), start_index_map=(0,))


def _lane_broadcast(vec, i):
  idx = jnp.zeros((16, 1), jnp.int32) + i
  return lax.gather(
      vec, idx, _GATHER_DNUMS, (1,),
      mode=lax.GatherScatterMode.PROMISE_IN_BOUNDS)


def _make_kernel(num_bins):
  assert num_bins % (_NUM_WORKERS * _GROUP_BINS) == 0
  bins_per_worker = num_bins // _NUM_WORKERS
  num_groups = bins_per_worker // _GROUP_BINS
  assert num_groups % 2 == 0 and num_groups >= 4

  mesh = plsc.VectorSubcoreMesh(
      core_axis_name="c",
      subcore_axis_name="s",
      num_cores=_NUM_CORES,
      num_subcores=_NUM_SUBCORES,
  )

  @functools.partial(
      pl.kernel,
      mesh=mesh,
      compiler_params=pltpu.CompilerParams(
          needs_layout_passes=False, use_tc_tiling_on_sc=False),
      out_type=jax.ShapeDtypeStruct((num_bins * _HIDDEN, ), jnp.float32),
      scratch_types=[
          pltpu.VMEM((_GROUP_ROWS, ), jnp.int32),
          pltpu.VMEM((_GROUP_ROWS, ), jnp.int32),
          pltpu.VMEM((_GROUP_ROWS, _HIDDEN), jnp.float32),
          pltpu.VMEM((_GROUP_ROWS, _HIDDEN), jnp.float32),
          pltpu.VMEM((_GROUP_BINS * _HIDDEN, ), jnp.float32),
          pltpu.SemaphoreType.DMA,
          pltpu.SemaphoreType.DMA,
          pltpu.SemaphoreType.DMA,
          pltpu.SemaphoreType.DMA,
      ],
  )
  def embed(tok_hbm, table_hbm, out_hbm, tok_v0, tok_v1, rows_v0, rows_v1,
            out_v, tok_sem0, tok_sem1, rows_sem0, rows_sem1):
    wid = lax.axis_index("s") * _NUM_CORES + lax.axis_index("c")
    worker_tok0 = wid * bins_per_worker * _TOKENS_PER_BIN
    worker_out0 = wid * bins_per_worker * _HIDDEN
    lane = lax.iota(jnp.int32, 16)
    tok_refs = (tok_v0, tok_v1)
    rows_refs = (rows_v0, rows_v1)
    tok_sems = (tok_sem0, tok_sem1)
    rows_sems = (rows_sem0, rows_sem1)

    def fire_tok(k, buf):
      src = tok_hbm.at[pl.ds(worker_tok0 + k * _GROUP_ROWS, _GROUP_ROWS)]
      pltpu.async_copy(src, tok_refs[buf], tok_sems[buf])

    def drain_tok(buf):
      pltpu.make_async_copy(
          tok_hbm.at[pl.ds(0, _GROUP_ROWS)], tok_refs[buf],
          tok_sems[buf]).wait()

    def fire_gathers(buf):
      for j in range(_IDX_CHUNKS):
        pltpu.async_copy(
            table_hbm.at[tok_refs[buf].at[pl.ds(j * _IDX_MINOR, _IDX_MINOR)]],
            rows_refs[buf].at[pl.ds(j * _IDX_MINOR, _IDX_MINOR)],
            rows_sems[buf],
        )

    def drain_rows(buf):
      pltpu.make_async_copy(
          table_hbm.at[pl.ds(0, _GROUP_ROWS)], rows_refs[buf],
          rows_sems[buf]).wait()

    def phase(g, cur, fire_next_tok, fire_next_gather):
      nxt = 1 - cur
      rows_cur = rows_refs[cur]
      drain_rows(cur)
      if fire_next_gather:
        drain_tok(nxt)
      invs = []
      for h2 in range(_GROUP_BINS // 16):
        binv = lane + h2 * 16
        cnt = jnp.zeros((16,), jnp.int32)
        for l in range(_TOKENS_PER_BIN):
          tv = plsc.load_gather(
              tok_refs[cur], [binv * _TOKENS_PER_BIN + l])
          cnt = cnt + jnp.where(tv != 0, jnp.int32(1), jnp.int32(0))
        invs.append(1.0 / jnp.maximum(cnt, 1).astype(jnp.float32))
      if fire_next_tok:
        fire_tok(g + 2, cur)
      if fire_next_gather:
        fire_gathers(nxt)

      def bin_body(b, carry):
        row0 = b * _TOKENS_PER_BIN
        acc = [rows_cur[row0, pl.ds(h * 16, 16)] for h in range(_H_CHUNKS)]
        for l in range(1, _TOKENS_PER_BIN):
          r = row0 + l
          for h in range(_H_CHUNKS):
            acc[h] = acc[h] + rows_cur[r, pl.ds(h * 16, 16)]
        inv = jnp.where(
            b < 16,
            _lane_broadcast(invs[0], jnp.minimum(b, 15)),
            _lane_broadcast(invs[1], jnp.maximum(b - 16, 0)),
        )
        for h in range(_H_CHUNKS):
          out_v[pl.ds(b * _HIDDEN + h * 16, 16)] = acc[h] * inv
        return carry

      lax.fori_loop(0, _GROUP_BINS, bin_body, 0)

      pltpu.sync_copy(
          out_v,
          out_hbm.at[pl.ds(worker_out0 + g * _GROUP_BINS * _HIDDEN,
                           _GROUP_BINS * _HIDDEN)],
      )

    fire_tok(0, 0)
    fire_tok(1, 1)
    drain_tok(0)
    fire_gathers(0)

    def pair_body(g2, carry):
      g = g2 * 2
      phase(g, 0, True, True)
      phase(g + 1, 1, True, True)
      return carry

    lax.fori_loop(0, num_groups // 2 - 1, pair_body, 0)
    phase(num_groups - 2, 0, False, True)
    phase(num_groups - 1, 1, False, False)

  return embed


def kernel(tokens, table):
  assert tokens.ndim == 3 and table.ndim == 2
  batch, feats, tpb = tokens.shape
  assert tpb == _TOKENS_PER_BIN and table.shape[1] == _HIDDEN
  num_bins = batch * feats
  tok_flat = tokens.astype(jnp.int32).reshape(-1)
  out = _make_kernel(num_bins)(tok_flat, table)
  return out.reshape(batch, feats, _HIDDEN)
